# Initial kernel scaffold; baseline (speedup 1.0000x reference)
import jax
import jax.numpy as jnp
from jax.experimental import pallas as pl


def kernel(pconf, gconf):
    raise NotImplementedError("write your pallas kernel here")


# speedup vs baseline: 4.9431x; 4.9431x over previous
import functools

import jax
import jax.numpy as jnp
import numpy as np
from jax.experimental import pallas as pl

NUM_BINS = 30
EPS_CLIP = 1e-7
_F16_EPS = float(np.finfo(np.float16).eps)
_SCALE = NUM_BINS - _F16_EPS
_PAD = 32


def _hist_body(p_ref, t_ref, hist_ref):
    i = pl.program_id(0)
    g = jnp.abs(p_ref[...] - t_ref[...])
    x = g * _SCALE
    parts = []
    for b in range(NUM_BINS):
        parts.append(jnp.sum((x >= np.float32(b)).astype(jnp.float32)))
    parts.append(jnp.float32(0.0))
    counts = [parts[b] - parts[b + 1] for b in range(NUM_BINS)]
    counts += [jnp.float32(0.0)] * (_PAD - NUM_BINS)
    h = jnp.stack(counts).reshape(1, _PAD)

    @pl.when(i == 0)
    def _():
        hist_ref[...] = h

    @pl.when(i > 0)
    def _():
        hist_ref[...] += h


def _loss_body(num_calc, hist_ref, p_ref, t_ref, out_ref):
    counts = hist_ref[...]
    nvalid = jnp.sum((counts > 0).astype(jnp.float32))
    scale = num_calc * nvalid
    recip = jnp.where(counts > 0, scale / jnp.maximum(counts, 1.0), 0.0)

    p = p_ref[...]
    t = t_ref[...]
    x = jnp.abs(p - t) * _SCALE
    w = jnp.full(p.shape, recip[0, 0], dtype=jnp.float32)
    for b in range(1, NUM_BINS):
        w = jnp.where(x >= np.float32(b), recip[0, b], w)

    pc = jnp.clip(p, EPS_CLIP, 1.0 - EPS_CLIP)
    bce = -(t * jnp.log(pc) + (1.0 - t) * jnp.log(1.0 - pc))
    out_ref[...] = bce * w


def kernel(pconf, gconf):
    m, n = pconf.shape
    bm = 256
    grid = (m // bm,)

    blk = pl.BlockSpec((bm, n), lambda i: (i, 0))
    hist = pl.pallas_call(
        _hist_body,
        grid=grid,
        in_specs=[blk, blk],
        out_specs=pl.BlockSpec((1, _PAD), lambda i: (0, 0)),
        out_shape=jax.ShapeDtypeStruct((1, _PAD), jnp.float32),
    )(pconf, gconf)

    num_calc = np.float32(m * n)
    loss = pl.pallas_call(
        functools.partial(_loss_body, num_calc),
        grid=grid,
        in_specs=[pl.BlockSpec((1, _PAD), lambda i: (0, 0)), blk, blk],
        out_specs=blk,
        out_shape=jax.ShapeDtypeStruct((m, n), jnp.float32),
    )(hist, pconf, gconf)
    return loss


# speedup vs baseline: 8.3344x; 1.6861x over previous
import functools

import jax
import jax.numpy as jnp
import numpy as np
from jax.experimental import pallas as pl

NUM_BINS = 30
EPS_CLIP = 1e-7
_F16_EPS = float(np.finfo(np.float16).eps)
_SCALE = NUM_BINS - _F16_EPS
_PAD = 32


def _hist_body(p_ref, t_ref, hist_ref):
    i = pl.program_id(0)
    g = jnp.abs(p_ref[...] - t_ref[...])
    x = g * _SCALE
    parts = []
    for b in range(NUM_BINS):
        parts.append(jnp.sum((x >= np.float32(b)).astype(jnp.float32)))
    parts.append(jnp.float32(0.0))
    counts = [parts[b] - parts[b + 1] for b in range(NUM_BINS)]
    counts += [jnp.float32(0.0)] * (_PAD - NUM_BINS)
    h = jnp.stack(counts).reshape(1, _PAD)

    @pl.when(i == 0)
    def _():
        hist_ref[...] = h

    @pl.when(i > 0)
    def _():
        hist_ref[...] += h


def _loss_body(num_calc, hist_ref, p_ref, t_ref, out_ref):
    counts = hist_ref[...]
    nvalid = jnp.sum((counts > 0).astype(jnp.float32))
    scale = num_calc * nvalid
    recip = jnp.where(counts > 0, scale / jnp.maximum(counts, 1.0), 0.0)

    p = p_ref[...]
    t = t_ref[...]
    bm, bn = p.shape
    x = jnp.abs(p - t) * _SCALE
    idx = x.astype(jnp.int32)
    table = jnp.concatenate(
        [recip, jnp.zeros((1, 128 - _PAD), jnp.float32)], axis=1
    )
    table = jnp.broadcast_to(table, (bm, 128))
    cols = []
    for k in range(bn // 128):
        idxk = idx[:, k * 128 : (k + 1) * 128]
        cols.append(
            jnp.take_along_axis(table, idxk, axis=1, mode="promise_in_bounds")
        )
    w = jnp.concatenate(cols, axis=1)

    pc = jnp.clip(p, EPS_CLIP, 1.0 - EPS_CLIP)
    bce = -(t * jnp.log(pc) + (1.0 - t) * jnp.log(1.0 - pc))
    out_ref[...] = bce * w


def kernel(pconf, gconf):
    m, n = pconf.shape
    bm = 256
    grid = (m // bm,)

    blk = pl.BlockSpec((bm, n), lambda i: (i, 0))
    hist = pl.pallas_call(
        _hist_body,
        grid=grid,
        in_specs=[blk, blk],
        out_specs=pl.BlockSpec((1, _PAD), lambda i: (0, 0)),
        out_shape=jax.ShapeDtypeStruct((1, _PAD), jnp.float32),
    )(pconf, gconf)

    num_calc = np.float32(m * n)
    loss = pl.pallas_call(
        functools.partial(_loss_body, num_calc),
        grid=grid,
        in_specs=[pl.BlockSpec((1, _PAD), lambda i: (0, 0)), blk, blk],
        out_specs=blk,
        out_shape=jax.ShapeDtypeStruct((m, n), jnp.float32),
    )(hist, pconf, gconf)
    return loss
